# Initial kernel scaffold; baseline (speedup 1.0000x reference)
import jax
import jax.numpy as jnp
from jax.experimental import pallas as pl


def kernel(depth, rpe_table):
    raise NotImplementedError("write your pallas kernel here")


# speedup vs baseline: 41.1440x; 41.1440x over previous
import functools

import jax
import jax.numpy as jnp
from jax import lax
from jax.experimental import pallas as pl
from jax.experimental.pallas import tpu as pltpu
from jax.experimental.pallas import tpu_sc as plsc

H = 96
W = 96
NH = 16
P = 24
NB = 2 * P + 1
NC = 2
NS = 16
NW = NC * NS
IPW = H // NW
L = 16
JC = W // L
OSTRIDE = 384


def _round_clip(v):
    c = jnp.minimum(jnp.maximum(v, -24.0), 24.0)
    r = c + jnp.sign(c) * 0.5
    return r.astype(jnp.int32) + P


def _rpe_body(depth_hbm, tz_hbm, txy_hbm, out_hbm,
              dep_v, zn_v, tz_v, txy_v, kx_v, obuf_v, sem):
    wid = lax.axis_index("s") * NC + lax.axis_index("c")
    i0 = wid * IPW

    pltpu.sync_copy(depth_hbm, dep_v)
    pltpu.sync_copy(tz_hbm, tz_v)
    pltpu.sync_copy(txy_hbm, txy_v)

    def mm_body(c, carry):
        mn, mx = carry
        v = dep_v[pl.ds(c * L, L)]
        return jnp.minimum(mn, v), jnp.maximum(mx, v)

    first = dep_v[pl.ds(0, L)]
    mn, mx = lax.fori_loop(1, H * W // L, mm_body, (first, first))
    m_s = mn[0]
    x_s = mx[0]
    for k in range(1, L):
        m_s = jnp.minimum(m_s, mn[k])
        x_s = jnp.maximum(x_s, mx[k])
    r_s = (x_s - m_s) + jnp.float32(1e-8)

    def zn_body(c, _):
        zn_v[pl.ds(c * L, L)] = (dep_v[pl.ds(c * L, L)] - m_s) / r_s
        return 0

    lax.fori_loop(0, H * W // L, zn_body, 0)

    lane = lax.iota(jnp.int32, L)
    for il in range(IPW):
        xi = jnp.full((L,), i0 + il, jnp.int32).astype(jnp.float32)
        xi = xi / jnp.float32(W - 1)
        for jc in range(JC):
            xj = (lane + jc * L).astype(jnp.float32) / jnp.float32(W - 1)
            kx_v[pl.ds(il * W + jc * L, L)] = _round_clip((xi - xj) * 24.0)

    def h_body(h, _):
        for il in range(IPW):
            ia = jnp.full((L,), h * W + i0 + il, jnp.int32)
            za = plsc.load_gather(zn_v, [ia])
            for jc in range(JC):
                zb = zn_v[pl.ds(h * W + jc * L, L)]
                kz = _round_clip((za - zb) * 24.0)
                kx = kx_v[pl.ds(il * W + jc * L, L)]
                for n in range(NH):
                    a = plsc.load_gather(tz_v, [kz + n * NB])
                    b = plsc.load_gather(txy_v, [kx + n * NB])
                    obuf_v[pl.ds(n * OSTRIDE + il * W + jc * L, L)] = a + b
        descs = [
            pltpu.async_copy(
                obuf_v.at[pl.ds(n * OSTRIDE, IPW * W)],
                out_hbm.at[pl.ds(((h * NH + n) * W + i0) * W, IPW * W)],
                sem)
            for n in range(NH)
        ]
        for d in descs:
            d.wait()
        return 0

    lax.fori_loop(0, H, h_body, 0)


@jax.jit
def _rpe_sc(dep_flat, tz_t, txy_t):
    mesh = plsc.VectorSubcoreMesh(core_axis_name="c", subcore_axis_name="s",
                                  num_cores=NC, num_subcores=NS)
    return pl.kernel(
        _rpe_body,
        out_type=jax.ShapeDtypeStruct((H * NH * W * W,), jnp.float32),
        mesh=mesh,
        compiler_params=pltpu.CompilerParams(needs_layout_passes=False),
        scratch_types=[
            pltpu.VMEM((H * W,), jnp.float32),
            pltpu.VMEM((H * W,), jnp.float32),
            pltpu.VMEM((NH * NB,), jnp.float32),
            pltpu.VMEM((NH * NB,), jnp.float32),
            pltpu.VMEM((IPW * W,), jnp.int32),
            pltpu.VMEM((NH * OSTRIDE,), jnp.float32),
            pltpu.SemaphoreType.DMA,
        ],
    )(dep_flat, tz_t, txy_t)


def kernel(depth, rpe_table):
    dep_flat = depth.reshape(-1)
    tz_t = rpe_table[2 * NB:3 * NB, :].T.reshape(-1)
    txy_t = (rpe_table[0:NB, :] + rpe_table[NB + P, :]).T.reshape(-1)
    return _rpe_sc(dep_flat, tz_t, txy_t).reshape(H, NH, W, W)


# speedup vs baseline: 49.0711x; 1.1927x over previous
import functools

import jax
import jax.numpy as jnp
from jax import lax
from jax.experimental import pallas as pl
from jax.experimental.pallas import tpu as pltpu
from jax.experimental.pallas import tpu_sc as plsc

H = 96
W = 96
NH = 16
P = 24
NB = 2 * P + 1
NC = 2
NS = 16
NW = NC * NS
IPW = H // NW
L = 16
JC = W // L
OSTRIDE = 384
TPAD = 128


def _round_clip(v):
    c = jnp.minimum(jnp.maximum(v, -24.0), 24.0)
    r = c + jnp.sign(c) * 0.5
    return r.astype(jnp.int32) + P


def _rpe_body(depth_hbm, tz_hbm, txy_hbm, out_hbm,
              dep_v, zn_v, tz_v, txy_v, obufA, obufB, semA, semB):
    wid = lax.axis_index("s") * NC + lax.axis_index("c")
    i0 = wid * IPW

    pltpu.sync_copy(depth_hbm, dep_v)
    pltpu.sync_copy(tz_hbm, tz_v)
    pltpu.sync_copy(txy_hbm, txy_v)

    def mm_body(c, carry):
        mn, mx = carry
        v = dep_v[pl.ds(c * L, L)]
        return jnp.minimum(mn, v), jnp.maximum(mx, v)

    first = dep_v[pl.ds(0, L)]
    mn, mx = lax.fori_loop(1, H * W // L, mm_body, (first, first))
    m_s = mn[0]
    x_s = mx[0]
    for k in range(1, L):
        m_s = jnp.minimum(m_s, mn[k])
        x_s = jnp.maximum(x_s, mx[k])
    r_s = (x_s - m_s) + jnp.float32(1e-8)

    def zn_body(c, _):
        zn_v[pl.ds(c * L, L)] = (dep_v[pl.ds(c * L, L)] - m_s) / r_s
        return 0

    lax.fori_loop(0, H * W // L, zn_body, 0)

    lane = lax.iota(jnp.int32, L)
    kxs = []
    for il in range(IPW):
        xi = jnp.full((L,), i0 + il, jnp.int32).astype(jnp.float32)
        xi = xi / jnp.float32(W - 1)
        kxs.append([])
        for jc in range(JC):
            xj = (lane + jc * L).astype(jnp.float32) / jnp.float32(W - 1)
            kxs[il].append(_round_clip((xi - xj) * 24.0))

    def compute_h(h, obuf):
        zbs = [zn_v[pl.ds(h * W + jc * L, L)] for jc in range(JC)]
        for il in range(IPW):
            ia = jnp.full((L,), h * W + i0 + il, jnp.int32)
            za = plsc.load_gather(zn_v, [ia])
            for jc in range(JC):
                kz = _round_clip((za - zbs[jc]) * 24.0)
                kx = kxs[il][jc]
                for n in range(NH):
                    a = plsc.load_gather(tz_v.at[pl.ds(n * TPAD, TPAD)], [kz])
                    b = plsc.load_gather(txy_v.at[pl.ds(n * TPAD, TPAD)], [kx])
                    obuf[pl.ds(n * OSTRIDE + il * W + jc * L, L)] = a + b

    def fire(h, obuf, sem):
        for n in range(NH):
            pltpu.async_copy(
                obuf.at[pl.ds(n * OSTRIDE, IPW * W)],
                out_hbm.at[pl.ds(((h * NH + n) * W + i0) * W, IPW * W)],
                sem)

    def drain(obuf, sem):
        for n in range(NH):
            pltpu.make_async_copy(
                obuf.at[pl.ds(n * OSTRIDE, IPW * W)],
                out_hbm.at[pl.ds(0, IPW * W)],
                sem).wait()

    def h_body(hh, _):
        h0 = hh * 2

        @pl.when(hh > 0)
        def _():
            drain(obufA, semA)

        compute_h(h0, obufA)
        fire(h0, obufA, semA)

        @pl.when(hh > 0)
        def _():
            drain(obufB, semB)

        compute_h(h0 + 1, obufB)
        fire(h0 + 1, obufB, semB)
        return 0

    lax.fori_loop(0, H // 2, h_body, 0)
    drain(obufA, semA)
    drain(obufB, semB)


@jax.jit
def _rpe_sc(dep_flat, tz_t, txy_t):
    mesh = plsc.VectorSubcoreMesh(core_axis_name="c", subcore_axis_name="s",
                                  num_cores=NC, num_subcores=NS)
    return pl.kernel(
        _rpe_body,
        out_type=jax.ShapeDtypeStruct((H * NH * W * W,), jnp.float32),
        mesh=mesh,
        compiler_params=pltpu.CompilerParams(needs_layout_passes=False),
        scratch_types=[
            pltpu.VMEM((H * W,), jnp.float32),
            pltpu.VMEM((H * W,), jnp.float32),
            pltpu.VMEM((NH * TPAD,), jnp.float32),
            pltpu.VMEM((NH * TPAD,), jnp.float32),
            pltpu.VMEM((NH * OSTRIDE,), jnp.float32),
            pltpu.VMEM((NH * OSTRIDE,), jnp.float32),
            pltpu.SemaphoreType.DMA,
            pltpu.SemaphoreType.DMA,
        ],
    )(dep_flat, tz_t, txy_t)


def kernel(depth, rpe_table):
    dep_flat = depth.reshape(-1)
    tz_t = jnp.pad(rpe_table[2 * NB:3 * NB, :].T,
                   ((0, 0), (0, TPAD - NB))).reshape(-1)
    txy_t = jnp.pad((rpe_table[0:NB, :] + rpe_table[NB + P, :]).T,
                    ((0, 0), (0, TPAD - NB))).reshape(-1)
    return _rpe_sc(dep_flat, tz_t, txy_t).reshape(H, NH, W, W)


# speedup vs baseline: 60.6830x; 1.2366x over previous
import functools

import jax
import jax.numpy as jnp
from jax import lax
from jax.experimental import pallas as pl
from jax.experimental.pallas import tpu as pltpu
from jax.experimental.pallas import tpu_sc as plsc

H = 96
W = 96
NH = 16
P = 24
NB = 2 * P + 1
NC = 2
NS = 16
NW = NC * NS
IPW = H // NW
L = 16
JC = W // L
OSTRIDE = 384
TPAD = 128
KZS = 64
NBK = KZS * KZS


def _round_clip(v):
    c = jnp.minimum(jnp.maximum(v, -24.0), 24.0)
    r = c + jnp.sign(c) * 0.5
    return r.astype(jnp.int32) + P


def _rpe_body(depth_hbm, tz_hbm, txy_hbm, out_hbm,
              dep_v, zn_v, tz_v, txy_v, comb_v, obufA, obufB, semA, semB):
    wid = lax.axis_index("s") * NC + lax.axis_index("c")
    i0 = wid * IPW

    pltpu.sync_copy(depth_hbm, dep_v)
    pltpu.sync_copy(tz_hbm, tz_v)
    pltpu.sync_copy(txy_hbm, txy_v)

    def mm_body(c, carry):
        mn, mx = carry
        v = dep_v[pl.ds(c * L, L)]
        return jnp.minimum(mn, v), jnp.maximum(mx, v)

    first = dep_v[pl.ds(0, L)]
    mn, mx = lax.fori_loop(1, H * W // L, mm_body, (first, first))
    m_s = mn[0]
    x_s = mx[0]
    for k in range(1, L):
        m_s = jnp.minimum(m_s, mn[k])
        x_s = jnp.maximum(x_s, mx[k])
    r_s = (x_s - m_s) + jnp.float32(1e-8)

    def zn_body(c, _):
        zn_v[pl.ds(c * L, L)] = (dep_v[pl.ds(c * L, L)] - m_s) / r_s
        return 0

    lax.fori_loop(0, H * W // L, zn_body, 0)

    for n in range(NH):
        tzrow = [tz_v[pl.ds(n * TPAD + c * L, L)] for c in range(KZS // L)]

        def kx_body(kx, _, n=n, tzrow=tzrow):
            s = jnp.full((L,), n * TPAD, jnp.int32) + kx
            tv = plsc.load_gather(txy_v, [s])
            base = n * NBK + kx * KZS
            for c in range(KZS // L):
                comb_v[pl.ds(base + c * L, L)] = tv + tzrow[c]
            return 0

        lax.fori_loop(0, NB, kx_body, 0)

    lane = lax.iota(jnp.int32, L)
    kxs = []
    for il in range(IPW):
        xi = jnp.full((L,), i0 + il, jnp.int32).astype(jnp.float32)
        xi = xi / jnp.float32(W - 1)
        kxs.append([])
        for jc in range(JC):
            xj = (lane + jc * L).astype(jnp.float32) / jnp.float32(W - 1)
            kxs[il].append(_round_clip((xi - xj) * 24.0) * KZS)

    def compute_h(h, obuf):
        zbs = [zn_v[pl.ds(h * W + jc * L, L)] for jc in range(JC)]
        for il in range(IPW):
            ia = jnp.full((L,), h * W + i0 + il, jnp.int32)
            za = plsc.load_gather(zn_v, [ia])
            for jc in range(JC):
                ib = kxs[il][jc] + _round_clip((za - zbs[jc]) * 24.0)
                for n in range(NH):
                    v = plsc.load_gather(comb_v.at[pl.ds(n * NBK, NBK)], [ib])
                    obuf[pl.ds(n * OSTRIDE + il * W + jc * L, L)] = v

    def fire(h, obuf, sem):
        for n in range(NH):
            pltpu.async_copy(
                obuf.at[pl.ds(n * OSTRIDE, IPW * W)],
                out_hbm.at[pl.ds(((h * NH + n) * W + i0) * W, IPW * W)],
                sem)

    def drain(obuf, sem):
        for n in range(NH):
            pltpu.make_async_copy(
                obuf.at[pl.ds(n * OSTRIDE, IPW * W)],
                out_hbm.at[pl.ds(0, IPW * W)],
                sem).wait()

    def h_body(hh, _):
        h0 = hh * 2

        @pl.when(hh > 0)
        def _():
            drain(obufA, semA)

        compute_h(h0, obufA)
        fire(h0, obufA, semA)

        @pl.when(hh > 0)
        def _():
            drain(obufB, semB)

        compute_h(h0 + 1, obufB)
        fire(h0 + 1, obufB, semB)
        return 0

    lax.fori_loop(0, H // 2, h_body, 0)
    drain(obufA, semA)
    drain(obufB, semB)


@jax.jit
def _rpe_sc(dep_flat, tz_t, txy_t):
    mesh = plsc.VectorSubcoreMesh(core_axis_name="c", subcore_axis_name="s",
                                  num_cores=NC, num_subcores=NS)
    return pl.kernel(
        _rpe_body,
        out_type=jax.ShapeDtypeStruct((H * NH * W * W,), jnp.float32),
        mesh=mesh,
        compiler_params=pltpu.CompilerParams(needs_layout_passes=False),
        scratch_types=[
            pltpu.VMEM((H * W,), jnp.float32),
            pltpu.VMEM((H * W,), jnp.float32),
            pltpu.VMEM((NH * TPAD,), jnp.float32),
            pltpu.VMEM((NH * TPAD,), jnp.float32),
            pltpu.VMEM((NH * NBK,), jnp.float32),
            pltpu.VMEM((NH * OSTRIDE,), jnp.float32),
            pltpu.VMEM((NH * OSTRIDE,), jnp.float32),
            pltpu.SemaphoreType.DMA,
            pltpu.SemaphoreType.DMA,
        ],
    )(dep_flat, tz_t, txy_t)


def kernel(depth, rpe_table):
    dep_flat = depth.reshape(-1)
    tz_t = jnp.pad(rpe_table[2 * NB:3 * NB, :].T,
                   ((0, 0), (0, TPAD - NB))).reshape(-1)
    txy_t = jnp.pad((rpe_table[0:NB, :] + rpe_table[NB + P, :]).T,
                    ((0, 0), (0, TPAD - NB))).reshape(-1)
    return _rpe_sc(dep_flat, tz_t, txy_t).reshape(H, NH, W, W)
